# baseline (speedup 1.0000x reference)
import functools

import jax
import jax.numpy as jnp
from jax import lax
from jax.experimental import pallas as pl
from jax.experimental.pallas import tpu as pltpu
from jax.experimental.pallas import tpu_sc as plsc

NUM_EMB = 1000000
DIM = 64
N_TOTAL = 4096 * 200
NC, NS, L = 2, 16, 16
NW = NC * NS
PER_W = N_TOTAL // NW
TILE = 128
N_TILES = PER_W // TILE
BLK = TILE // L
NBUF = 4


def _vrsqrt(a):
    i = lax.bitcast_convert_type(a, jnp.int32)
    i = 0x5F3759DF - (i >> 1)
    y = lax.bitcast_convert_type(i, jnp.float32)
    half = a * 0.5
    for _ in range(2):
        y = y * (1.5 - half * y * y)
    return y


def _lane_shuffle(v, idx):
    return lax.gather(
        v, idx.reshape(L, 1),
        lax.GatherDimensionNumbers(offset_dims=(), collapsed_slice_dims=(0,),
                                   start_index_map=(0,)),
        slice_sizes=(1,),
        mode=lax.GatherScatterMode.PROMISE_IN_BOUNDS)


def _normalize_tile(rows):
    iota = lax.iota(jnp.int32, L)
    perms = [iota ^ sh for sh in (8, 4, 2, 1)]

    def blk_body(blk, _):
        for j in range(L):
            r = blk * L + j
            v = [rows[r, pl.ds(k * L, L)] for k in range(DIM // L)]
            s = (v[0] * v[0] + v[1] * v[1]) + (v[2] * v[2] + v[3] * v[3])
            for p in perms:
                s = s + _lane_shuffle(s, p)
            y = _vrsqrt(s * (1.0 / DIM) + 1e-6)
            for k in range(DIM // L):
                rows[r, pl.ds(k * L, L)] = v[k] * y
        return 0

    lax.fori_loop(0, BLK, blk_body, 0)


def _sc_body(w_hbm, xf_hbm, out_hbm, idx_v,
             buf0, buf1, buf2, buf3,
             sg0, sg1, sg2, sg3, so0, so1, so2, so3):
    wid = lax.axis_index("s") * NC + lax.axis_index("c")
    base = wid * PER_W
    pltpu.sync_copy(xf_hbm.at[pl.ds(base, PER_W)], idx_v)

    bufs = (buf0, buf1, buf2, buf3)
    sg = (sg0, sg1, sg2, sg3)
    so = (so0, so1, so2, so3)

    def start_gather(t, b):
        pltpu.async_copy(w_hbm.at[idx_v.at[pl.ds(t * TILE, TILE)]],
                         bufs[b], sg[b])

    def wait_gather(b):
        pltpu.make_async_copy(
            w_hbm.at[idx_v.at[pl.ds(0, TILE)]], bufs[b], sg[b]).wait()

    def wait_out(b):
        pltpu.make_async_copy(
            bufs[b], out_hbm.at[pl.ds(base, TILE)], so[b]).wait()

    for tt in range(NBUF - 1):
        start_gather(tt, tt)

    def quad_body(i, _):
        for b in range(NBUF):
            t = NBUF * i + b
            pb = (b + NBUF - 1) % NBUF

            @pl.when(t >= 1)
            def _():
                wait_out(pb)

            start_gather((t + NBUF - 1) % N_TILES, pb)
            wait_gather(b)
            _normalize_tile(bufs[b])
            pltpu.async_copy(bufs[b],
                             out_hbm.at[pl.ds(base + t * TILE, TILE)], so[b])
        return 0

    lax.fori_loop(0, N_TILES // NBUF, quad_body, 0)

    wait_out((N_TILES - 1) % NBUF)
    for b in range(NBUF - 1):
        wait_gather(b)


@jax.jit
def _sc_lookup(weight, xf):
    mesh = plsc.VectorSubcoreMesh(core_axis_name="c", subcore_axis_name="s")
    return pl.kernel(
        _sc_body,
        out_type=jax.ShapeDtypeStruct((N_TOTAL, DIM), jnp.float32),
        mesh=mesh,
        compiler_params=pltpu.CompilerParams(use_tc_tiling_on_sc=False,
                                             needs_layout_passes=False),
        scratch_types=(
            [pltpu.VMEM((PER_W,), jnp.int32)]
            + [pltpu.VMEM((TILE, DIM), jnp.float32)] * NBUF
            + [pltpu.SemaphoreType.DMA] * (2 * NBUF)
        ),
    )(weight, xf)


def kernel(x, weight):
    xf = x.reshape(-1).astype(jnp.int32)
    out = _sc_lookup(weight, xf)
    return out.reshape(x.shape + (DIM,))


# speedup vs baseline: 1.2438x; 1.2438x over previous
import functools

import jax
import jax.numpy as jnp
from jax import lax
from jax.experimental import pallas as pl
from jax.experimental.pallas import tpu as pltpu
from jax.experimental.pallas import tpu_sc as plsc

NUM_EMB = 1000000
DIM = 64
GRP = 128
N_TOTAL = 4096 * 200
NC, NS, L = 2, 16, 16
NW = NC * NS
PER_W = N_TOTAL // NW
TILE = 128
N_TILES = PER_W // TILE
BLK = TILE // L


def _vrsqrt(a):
    i = lax.bitcast_convert_type(a, jnp.int32)
    i = 0x5F3759DF - (i >> 1)
    y = lax.bitcast_convert_type(i, jnp.float32)
    half = a * 0.5
    for _ in range(2):
        y = y * (1.5 - half * y * y)
    return y


def _lane_shuffle(v, idx):
    return lax.gather(
        v, idx.reshape(L, 1),
        lax.GatherDimensionNumbers(offset_dims=(), collapsed_slice_dims=(0,),
                                   start_index_map=(0,)),
        slice_sizes=(1,),
        mode=lax.GatherScatterMode.PROMISE_IN_BOUNDS)


def _normalize_tile(t, idx_v, rows, orows):
    iota = lax.iota(jnp.int32, L)
    perms = [iota ^ sh for sh in (8, 4, 2, 1)]

    def blk_body(blk, _):
        pvec = idx_v[pl.ds(t * TILE + blk * L, L)] & 1
        for j in range(L):
            r = blk * L + j
            pj = _lane_shuffle(pvec, jnp.full((L,), j, jnp.int32)) > 0
            v = [jnp.where(pj,
                           rows[r, pl.ds(DIM + k * L, L)],
                           rows[r, pl.ds(k * L, L)])
                 for k in range(DIM // L)]
            s = (v[0] * v[0] + v[1] * v[1]) + (v[2] * v[2] + v[3] * v[3])
            for pm in perms:
                s = s + _lane_shuffle(s, pm)
            y = _vrsqrt(s * (1.0 / DIM) + 1e-6)
            for k in range(DIM // L):
                orows[r, pl.ds(k * L, L)] = v[k] * y
        return 0

    lax.fori_loop(0, BLK, blk_body, 0)


def _sc_body(w_hbm, xf_hbm, xfh_hbm, out_hbm,
             idx_v, idxh_v, buf0, buf1, ob0, ob1, sg0, sg1, so0, so1):
    wid = lax.axis_index("s") * NC + lax.axis_index("c")
    base = wid * PER_W
    pltpu.sync_copy(xf_hbm.at[pl.ds(base, PER_W)], idx_v)
    pltpu.sync_copy(xfh_hbm.at[pl.ds(base, PER_W)], idxh_v)

    bufs = (buf0, buf1)
    obufs = (ob0, ob1)
    sg = (sg0, sg1)
    so = (so0, so1)

    def start_gather(t, b):
        pltpu.async_copy(w_hbm.at[idxh_v.at[pl.ds(t * TILE, TILE)]],
                         bufs[b], sg[b])

    def wait_gather(b):
        pltpu.make_async_copy(
            w_hbm.at[idxh_v.at[pl.ds(0, TILE)]], bufs[b], sg[b]).wait()

    def wait_out(b):
        pltpu.make_async_copy(
            obufs[b], out_hbm.at[pl.ds(base, TILE)], so[b]).wait()

    start_gather(0, 0)

    def pair_body(i, _):
        for b in (0, 1):
            t = 2 * i + b
            nb = 1 - b

            start_gather((t + 1) % N_TILES, nb)
            wait_gather(b)

            @pl.when(t >= 2)
            def _():
                wait_out(b)

            _normalize_tile(t, idx_v, bufs[b], obufs[b])
            pltpu.async_copy(obufs[b],
                             out_hbm.at[pl.ds(base + t * TILE, TILE)], so[b])
        return 0

    lax.fori_loop(0, N_TILES // 2, pair_body, 0)

    wait_out(0)
    wait_out(1)
    wait_gather(0)


@jax.jit
def _sc_lookup(w2, xf, xfh):
    mesh = plsc.VectorSubcoreMesh(core_axis_name="c", subcore_axis_name="s")
    return pl.kernel(
        _sc_body,
        out_type=jax.ShapeDtypeStruct((N_TOTAL, DIM), jnp.float32),
        mesh=mesh,
        compiler_params=pltpu.CompilerParams(needs_layout_passes=False),
        scratch_types=(
            [pltpu.VMEM((PER_W,), jnp.int32),
             pltpu.VMEM((PER_W,), jnp.int32),
             pltpu.VMEM((TILE, GRP), jnp.float32),
             pltpu.VMEM((TILE, GRP), jnp.float32),
             pltpu.VMEM((TILE, DIM), jnp.float32),
             pltpu.VMEM((TILE, DIM), jnp.float32)]
            + [pltpu.SemaphoreType.DMA] * 4
        ),
    )(w2, xf, xfh)


def kernel(x, weight):
    xf = x.reshape(-1).astype(jnp.int32)
    w2 = weight.reshape(NUM_EMB * DIM // GRP, GRP)
    out = _sc_lookup(w2, xf, xf >> 1)
    return out.reshape(x.shape + (DIM,))
